# baseline (speedup 1.0000x reference)
import functools

import jax
import jax.numpy as jnp
from jax import lax
from jax.experimental import pallas as pl
from jax.experimental.pallas import tpu as pltpu
from jax.experimental.pallas import tpu_sc as plsc

B = 1024
P = 32
M = 32
V = 64
V2 = 32
NPS = 768
NMV = 384
NCK = NMV // 128
BT = 128
NHALF = 2
BH = B // NHALF

NC = 2
NS = 16
NW = NC * NS

_sc_mesh = functools.partial(
    plsc.VectorSubcoreMesh, core_axis_name="c", subcore_axis_name="s",
    num_cores=NC, num_subcores=NS)


def _make_embed(nb):
    pos_w = nb // NW
    idx_w = pos_w * P
    ich = 128
    nch = idx_w // ich
    pos_ch = ich // P

    @functools.partial(
        pl.kernel,
        mesh=_sc_mesh(),
        out_type=jax.ShapeDtypeStruct((nb, 128), jnp.float32),
        compiler_params=pltpu.CompilerParams(use_tc_tiling_on_sc=False),
        scratch_types=[
            pltpu.VMEM((nch, ich), jnp.int32),
            pltpu.VMEM((2, ich, V), jnp.float32),
            pltpu.VMEM((128,), jnp.float32),
            pltpu.VMEM((pos_w, 128), jnp.float32),
            pltpu.SemaphoreType.DMA,
            pltpu.SemaphoreType.DMA,
        ],
    )
    def _sc_embed(idx_hbm, psv_hbm, psb_hbm, out_hbm, idx_v, rows_v, psb_v,
                  acc_v, sem0, sem1):
        wid = lax.axis_index("s") * NC + lax.axis_index("c")
        pltpu.sync_copy(idx_hbm.at[wid], idx_v)
        pltpu.sync_copy(psb_hbm, psb_v)
        sems = (sem0, sem1)
        cps = [None, None]
        cps[0] = pltpu.async_copy(psv_hbm.at[idx_v.at[0]], rows_v.at[0],
                                  sems[0])
        for j in range(nch):
            sl = j % 2
            if j + 1 < nch:
                cps[1 - sl] = pltpu.async_copy(
                    psv_hbm.at[idx_v.at[j + 1]], rows_v.at[1 - sl],
                    sems[1 - sl])
            cps[sl].wait()

            def pos_body(k, carry, sl=sl, j=j):
                accs = [psb_v[pl.ds(c * 16, 16)] for c in range(8)]
                for r in range(P):
                    for c in range(V // 16):
                        accs[c] = accs[c] + rows_v[sl, k * P + r,
                                                   pl.ds(c * 16, 16)]
                for c in range(8):
                    acc_v[j * pos_ch + k, pl.ds(c * 16, 16)] = accs[c]
                return carry

            lax.fori_loop(0, pos_ch, pos_body, 0)
        pltpu.sync_copy(acc_v, out_hbm.at[pl.ds(wid * pos_w, pos_w)])

    return _sc_embed


def _make_pick(nb):
    pos_w = nb // NW
    row_w = pos_w * M // 128
    pos_row = 128 // M

    @functools.partial(
        pl.kernel,
        mesh=_sc_mesh(),
        out_type=jax.ShapeDtypeStruct((NW, row_w, 128), jnp.float32),
        compiler_params=pltpu.CompilerParams(use_tc_tiling_on_sc=False,
                                             needs_layout_passes=False),
        scratch_types=[
            pltpu.VMEM((row_w, 128), jnp.int32),
            pltpu.VMEM((NCK, pos_w, 128), jnp.float32),
            pltpu.VMEM((row_w, 128), jnp.float32),
        ],
    )
    def _sc_pick(midx_hbm, scores_hbm, out_hbm, midx_v, rows_v, out_v):
        wid = lax.axis_index("s") * NC + lax.axis_index("c")
        base = wid * pos_w
        pltpu.sync_copy(midx_hbm.at[wid], midx_v)
        for c in range(NCK):
            pltpu.sync_copy(scores_hbm.at[c, pl.ds(base, pos_w)],
                            rows_v.at[c])

        def row_body(r, carry):
            for q in range(8):
                cvec = midx_v[r, pl.ds(q * 16, 16)]
                ckvec = lax.shift_right_logical(cvec, 7)
                colvec = lax.bitwise_and(cvec, 127)
                posvec = jnp.broadcast_to(
                    r * pos_row + q // 2, (16,)).astype(jnp.int32)
                out_v[r, pl.ds(q * 16, 16)] = plsc.load_gather(
                    rows_v, [ckvec, posvec, colvec])
            return carry

        lax.fori_loop(0, row_w, row_body, 0)
        pltpu.sync_copy(out_v, out_hbm.at[wid])

    return _sc_pick


def _tc_body(b_ref, w_ref, ow_ref, ob_ref, o_ref):
    bvec = b_ref[...]
    acc = jnp.zeros((BT, NMV), jnp.float32) + ob_ref[...]
    for h in range(V2):
        hid = jnp.dot(bvec, w_ref[h], preferred_element_type=jnp.float32)
        acc = acc + jnp.maximum(hid, 0.0) * ow_ref[h][None, :]
    for c in range(NCK):
        o_ref[c] = acc[:, c * 128:(c + 1) * 128]


def _tc_dense(b128, w, ow, ob):
    nb = b128.shape[0]
    return pl.pallas_call(
        _tc_body,
        grid=(nb // BT,),
        in_specs=[
            pl.BlockSpec((BT, 128), lambda i: (i, 0)),
            pl.BlockSpec((V2, 128, NMV), lambda i: (0, 0, 0)),
            pl.BlockSpec((V2, NMV), lambda i: (0, 0)),
            pl.BlockSpec((1, NMV), lambda i: (0, 0)),
        ],
        out_specs=pl.BlockSpec((NCK, BT, 128), lambda i: (0, i, 0)),
        out_shape=jax.ShapeDtypeStruct((NCK, nb, 128), jnp.float32),
    )(b128, w, ow, ob)


_embed_h = _make_embed(BH)
_pick_h = _make_pick(BH)


@jax.jit
def kernel(piece_square_idx, move_idx, ps_vectors, move_vectors, ps_bias,
           bias2, output_layer, output_bias):
    nch_h = BH * P // NW // 128
    row_w_h = BH * M // NW // 128
    psq = piece_square_idx.astype(jnp.int32).reshape(NHALF, NW, nch_h, 128)
    midx = move_idx.astype(jnp.int32).reshape(NHALF, NW, row_w_h, 128)
    psb128 = jnp.concatenate(
        [ps_bias, jnp.ones((1,), jnp.float32), jnp.zeros((63,), jnp.float32)])
    w = jnp.transpose(move_vectors, (2, 1, 0))
    b2 = jnp.transpose(bias2, (1, 0))
    w_aug = jnp.concatenate(
        [w, b2[:, None, :], jnp.zeros((V2, 63, NMV), jnp.float32)], axis=1)
    ow = jnp.transpose(output_layer, (1, 0))
    ob = output_bias[None, :]

    outs = []
    bs = [_embed_h(psq[h], ps_vectors, psb128) for h in range(NHALF)]
    ss = [_tc_dense(bs[h], w_aug, ow, ob) for h in range(NHALF)]
    outs = [_pick_h(midx[h], ss[h]) for h in range(NHALF)]
    return jnp.concatenate(outs).reshape(B, M)


# speedup vs baseline: 1.0841x; 1.0841x over previous
import functools

import jax
import jax.numpy as jnp
from jax import lax
from jax.experimental import pallas as pl
from jax.experimental.pallas import tpu as pltpu
from jax.experimental.pallas import tpu_sc as plsc

B = 1024
P = 32
M = 32
V = 64
V2 = 32
NPS = 768
NMV = 384
NCK = NMV // 128
BT = 256

NC = 2
NS = 16
NW = NC * NS
POS_W = B // NW
IDX_W = POS_W * P
ICH = 128
NCH = IDX_W // ICH
ROW_W = IDX_W // 128

_sc_mesh = functools.partial(
    plsc.VectorSubcoreMesh, core_axis_name="c", subcore_axis_name="s",
    num_cores=NC, num_subcores=NS)


@functools.partial(
    pl.kernel,
    mesh=_sc_mesh(),
    out_type=jax.ShapeDtypeStruct((B, 128), jnp.float32),
    compiler_params=pltpu.CompilerParams(use_tc_tiling_on_sc=False),
    scratch_types=[
        pltpu.VMEM((NCH, ICH), jnp.int32),
        pltpu.VMEM((2, ICH, V), jnp.float32),
        pltpu.VMEM((128,), jnp.float32),
        pltpu.VMEM((POS_W, 128), jnp.float32),
        pltpu.SemaphoreType.DMA,
        pltpu.SemaphoreType.DMA,
    ],
)
def _sc_embed(idx_hbm, psv_hbm, psb_hbm, out_hbm, idx_v, rows_v, psb_v,
              acc_v, sem0, sem1):
    wid = lax.axis_index("s") * NC + lax.axis_index("c")
    pltpu.sync_copy(idx_hbm.at[wid], idx_v)
    pltpu.sync_copy(psb_hbm, psb_v)
    sems = (sem0, sem1)
    POS_CH = ICH // P
    cps = [None, None]
    cps[0] = pltpu.async_copy(psv_hbm.at[idx_v.at[0]], rows_v.at[0], sems[0])
    for j in range(NCH):
        sl = j % 2
        if j + 1 < NCH:
            cps[1 - sl] = pltpu.async_copy(
                psv_hbm.at[idx_v.at[j + 1]], rows_v.at[1 - sl], sems[1 - sl])
        cps[sl].wait()

        def pos_body(k, carry, sl=sl, j=j):
            accs = [psb_v[pl.ds(c * 16, 16)] for c in range(8)]
            for r in range(P):
                for c in range(V // 16):
                    accs[c] = accs[c] + rows_v[sl, k * P + r,
                                               pl.ds(c * 16, 16)]
            for c in range(8):
                acc_v[j * POS_CH + k, pl.ds(c * 16, 16)] = accs[c]
            return carry

        lax.fori_loop(0, POS_CH, pos_body, 0)
    pltpu.sync_copy(acc_v, out_hbm.at[pl.ds(wid * POS_W, POS_W)])


@functools.partial(
    pl.kernel,
    mesh=_sc_mesh(),
    out_type=jax.ShapeDtypeStruct((NW, ROW_W, 128), jnp.float32),
    compiler_params=pltpu.CompilerParams(use_tc_tiling_on_sc=False,
                                         needs_layout_passes=False),
    scratch_types=[
        pltpu.VMEM((ROW_W, 128), jnp.int32),
        pltpu.VMEM((NCK, POS_W, 128), jnp.float32),
        pltpu.VMEM((ROW_W, 128), jnp.float32),
    ],
)
def _sc_pick(midx_hbm, scores_hbm, out_hbm, midx_v, rows_v, out_v):
    wid = lax.axis_index("s") * NC + lax.axis_index("c")
    base = wid * POS_W
    pltpu.sync_copy(midx_hbm.at[wid], midx_v)
    for c in range(NCK):
        pltpu.sync_copy(scores_hbm.at[c, pl.ds(base, POS_W)], rows_v.at[c])

    def row_body(r, carry):
        for q in range(8):
            cvec = midx_v[r, pl.ds(q * 16, 16)]
            ckvec = lax.shift_right_logical(cvec, 7)
            colvec = lax.bitwise_and(cvec, 127)
            posvec = jnp.broadcast_to(r * 4 + q // 2, (16,)).astype(jnp.int32)
            out_v[r, pl.ds(q * 16, 16)] = plsc.load_gather(
                rows_v, [ckvec, posvec, colvec])
        return carry

    lax.fori_loop(0, ROW_W, row_body, 0)
    pltpu.sync_copy(out_v, out_hbm.at[wid])


def _tc_body(b_ref, w_ref, ow_ref, ob_ref, o_ref):
    bvec = b_ref[...]
    acc = jnp.zeros((BT, NMV), jnp.float32) + ob_ref[...]
    for h in range(V2):
        hid = jnp.dot(bvec, w_ref[h], preferred_element_type=jnp.float32)
        acc = acc + jnp.maximum(hid, 0.0) * ow_ref[h][None, :]
    for c in range(NCK):
        o_ref[c] = acc[:, c * 128:(c + 1) * 128]


def _tc_dense(b128, w, ow, ob):
    return pl.pallas_call(
        _tc_body,
        grid=(B // BT,),
        in_specs=[
            pl.BlockSpec((BT, 128), lambda i: (i, 0)),
            pl.BlockSpec((V2, 128, NMV), lambda i: (0, 0, 0)),
            pl.BlockSpec((V2, NMV), lambda i: (0, 0)),
            pl.BlockSpec((1, NMV), lambda i: (0, 0)),
        ],
        out_specs=pl.BlockSpec((NCK, BT, 128), lambda i: (0, i, 0)),
        out_shape=jax.ShapeDtypeStruct((NCK, B, 128), jnp.float32),
    )(b128, w, ow, ob)


@jax.jit
def kernel(piece_square_idx, move_idx, ps_vectors, move_vectors, ps_bias,
           bias2, output_layer, output_bias):
    psq = piece_square_idx.astype(jnp.int32).reshape(NW, NCH, ICH)
    midx = move_idx.astype(jnp.int32).reshape(NW, ROW_W, 128)
    psb128 = jnp.concatenate(
        [ps_bias, jnp.ones((1,), jnp.float32), jnp.zeros((63,), jnp.float32)])
    w = jnp.transpose(move_vectors, (2, 1, 0))
    b2 = jnp.transpose(bias2, (1, 0))
    w_aug = jnp.concatenate(
        [w, b2[:, None, :], jnp.zeros((V2, 63, NMV), jnp.float32)], axis=1)
    ow = jnp.transpose(output_layer, (1, 0))

    b128 = _sc_embed(psq, ps_vectors, psb128)
    scores = _tc_dense(b128, w_aug, ow, output_bias[None, :])
    return _sc_pick(midx, scores).reshape(B, M)


# speedup vs baseline: 1.0878x; 1.0035x over previous
import functools

import jax
import jax.numpy as jnp
from jax import lax
from jax.experimental import pallas as pl
from jax.experimental.pallas import tpu as pltpu
from jax.experimental.pallas import tpu_sc as plsc

B = 1024
P = 32
M = 32
V = 64
V2 = 32
NPS = 768
NMV = 384
NCK = NMV // 128
BT = 512

NC = 2
NS = 16
NW = NC * NS
POS_W = B // NW
IDX_W = POS_W * P
ICH = 128
NCH = IDX_W // ICH
ROW_W = IDX_W // 128

_sc_mesh = functools.partial(
    plsc.VectorSubcoreMesh, core_axis_name="c", subcore_axis_name="s",
    num_cores=NC, num_subcores=NS)


@functools.partial(
    pl.kernel,
    mesh=_sc_mesh(),
    out_type=jax.ShapeDtypeStruct((B, 128), jnp.float32),
    compiler_params=pltpu.CompilerParams(use_tc_tiling_on_sc=False),
    scratch_types=[
        pltpu.VMEM((NCH, ICH), jnp.int32),
        pltpu.VMEM((2, ICH, V), jnp.float32),
        pltpu.VMEM((128,), jnp.float32),
        pltpu.VMEM((POS_W, 128), jnp.float32),
        pltpu.SemaphoreType.DMA,
        pltpu.SemaphoreType.DMA,
    ],
)
def _sc_embed(idx_hbm, psv_hbm, psb_hbm, out_hbm, idx_v, rows_v, psb_v,
              acc_v, sem0, sem1):
    wid = lax.axis_index("s") * NC + lax.axis_index("c")
    pltpu.sync_copy(idx_hbm.at[wid], idx_v)
    pltpu.sync_copy(psb_hbm, psb_v)
    sems = (sem0, sem1)
    POS_CH = ICH // P
    cps = [None, None]
    cps[0] = pltpu.async_copy(psv_hbm.at[idx_v.at[0]], rows_v.at[0], sems[0])
    for j in range(NCH):
        sl = j % 2
        if j + 1 < NCH:
            cps[1 - sl] = pltpu.async_copy(
                psv_hbm.at[idx_v.at[j + 1]], rows_v.at[1 - sl], sems[1 - sl])
        cps[sl].wait()

        def pos_body(k, carry, sl=sl, j=j):
            accs = [psb_v[pl.ds(c * 16, 16)] for c in range(8)]
            for r in range(P):
                for c in range(V // 16):
                    accs[c] = accs[c] + rows_v[sl, k * P + r,
                                               pl.ds(c * 16, 16)]
            for c in range(8):
                acc_v[j * POS_CH + k, pl.ds(c * 16, 16)] = accs[c]
            return carry

        lax.fori_loop(0, POS_CH, pos_body, 0)
    pltpu.sync_copy(acc_v, out_hbm.at[pl.ds(wid * POS_W, POS_W)])


@functools.partial(
    pl.kernel,
    mesh=_sc_mesh(),
    out_type=jax.ShapeDtypeStruct((NW, ROW_W, 128), jnp.float32),
    compiler_params=pltpu.CompilerParams(use_tc_tiling_on_sc=False,
                                         needs_layout_passes=False),
    scratch_types=[
        pltpu.VMEM((ROW_W, 128), jnp.int32),
        pltpu.VMEM((NCK, POS_W, 128), jnp.float32),
        pltpu.VMEM((ROW_W, 128), jnp.float32),
    ],
)
def _sc_pick(midx_hbm, scores_hbm, out_hbm, midx_v, rows_v, out_v):
    wid = lax.axis_index("s") * NC + lax.axis_index("c")
    base = wid * POS_W
    pltpu.sync_copy(midx_hbm.at[wid], midx_v)
    for c in range(NCK):
        pltpu.sync_copy(scores_hbm.at[c, pl.ds(base, POS_W)], rows_v.at[c])

    def row_body(r, carry):
        for q in range(8):
            cvec = midx_v[r, pl.ds(q * 16, 16)]
            ckvec = lax.shift_right_logical(cvec, 7)
            colvec = lax.bitwise_and(cvec, 127)
            posvec = jnp.broadcast_to(r * 4 + q // 2, (16,)).astype(jnp.int32)
            out_v[r, pl.ds(q * 16, 16)] = plsc.load_gather(
                rows_v, [ckvec, posvec, colvec])
        return carry

    lax.fori_loop(0, ROW_W, row_body, 0)
    pltpu.sync_copy(out_v, out_hbm.at[wid])


def _tc_body(b_ref, w_ref, ow_ref, ob_ref, o_ref):
    bvec = b_ref[...]
    acc = jnp.zeros((BT, NMV), jnp.float32) + ob_ref[...]
    for h in range(V2):
        hid = jnp.dot(bvec, w_ref[h], preferred_element_type=jnp.float32)
        acc = acc + jnp.maximum(hid, 0.0) * ow_ref[h][None, :]
    for c in range(NCK):
        o_ref[c] = acc[:, c * 128:(c + 1) * 128]


def _tc_dense(b128, w, ow, ob):
    return pl.pallas_call(
        _tc_body,
        grid=(B // BT,),
        in_specs=[
            pl.BlockSpec((BT, 128), lambda i: (i, 0)),
            pl.BlockSpec((V2, 128, NMV), lambda i: (0, 0, 0)),
            pl.BlockSpec((V2, NMV), lambda i: (0, 0)),
            pl.BlockSpec((1, NMV), lambda i: (0, 0)),
        ],
        out_specs=pl.BlockSpec((NCK, BT, 128), lambda i: (0, i, 0)),
        out_shape=jax.ShapeDtypeStruct((NCK, B, 128), jnp.float32),
    )(b128, w, ow, ob)


@jax.jit
def kernel(piece_square_idx, move_idx, ps_vectors, move_vectors, ps_bias,
           bias2, output_layer, output_bias):
    psq = piece_square_idx.astype(jnp.int32).reshape(NW, NCH, ICH)
    midx = move_idx.astype(jnp.int32).reshape(NW, ROW_W, 128)
    psb128 = jnp.concatenate(
        [ps_bias, jnp.ones((1,), jnp.float32), jnp.zeros((63,), jnp.float32)])
    w = jnp.transpose(move_vectors, (2, 1, 0))
    b2 = jnp.transpose(bias2, (1, 0))
    w_aug = jnp.concatenate(
        [w, b2[:, None, :], jnp.zeros((V2, 63, NMV), jnp.float32)], axis=1)
    ow = jnp.transpose(output_layer, (1, 0))

    b128 = _sc_embed(psq, ps_vectors, psb128)
    scores = _tc_dense(b128, w_aug, ow, output_bias[None, :])
    return _sc_pick(midx, scores).reshape(B, M)


# speedup vs baseline: 1.0886x; 1.0007x over previous
import functools

import jax
import jax.numpy as jnp
from jax import lax
from jax.experimental import pallas as pl
from jax.experimental.pallas import tpu as pltpu
from jax.experimental.pallas import tpu_sc as plsc

B = 1024
P = 32
M = 32
V = 64
V2 = 32
NPS = 768
NMV = 384
NCK = NMV // 128
BT = 1024

NC = 2
NS = 16
NW = NC * NS
POS_W = B // NW
IDX_W = POS_W * P
ICH = 128
NCH = IDX_W // ICH
ROW_W = IDX_W // 128

_sc_mesh = functools.partial(
    plsc.VectorSubcoreMesh, core_axis_name="c", subcore_axis_name="s",
    num_cores=NC, num_subcores=NS)


@functools.partial(
    pl.kernel,
    mesh=_sc_mesh(),
    out_type=jax.ShapeDtypeStruct((B, 128), jnp.float32),
    compiler_params=pltpu.CompilerParams(use_tc_tiling_on_sc=False),
    scratch_types=[
        pltpu.VMEM((NCH, ICH), jnp.int32),
        pltpu.VMEM((2, ICH, V), jnp.float32),
        pltpu.VMEM((128,), jnp.float32),
        pltpu.VMEM((POS_W, 128), jnp.float32),
        pltpu.SemaphoreType.DMA,
        pltpu.SemaphoreType.DMA,
    ],
)
def _sc_embed(idx_hbm, psv_hbm, psb_hbm, out_hbm, idx_v, rows_v, psb_v,
              acc_v, sem0, sem1):
    wid = lax.axis_index("s") * NC + lax.axis_index("c")
    pltpu.sync_copy(idx_hbm.at[wid], idx_v)
    pltpu.sync_copy(psb_hbm, psb_v)
    sems = (sem0, sem1)
    POS_CH = ICH // P
    cps = [None, None]
    cps[0] = pltpu.async_copy(psv_hbm.at[idx_v.at[0]], rows_v.at[0], sems[0])
    for j in range(NCH):
        sl = j % 2
        if j + 1 < NCH:
            cps[1 - sl] = pltpu.async_copy(
                psv_hbm.at[idx_v.at[j + 1]], rows_v.at[1 - sl], sems[1 - sl])
        cps[sl].wait()

        def pos_body(k, carry, sl=sl, j=j):
            accs = [psb_v[pl.ds(c * 16, 16)] for c in range(8)]
            for r in range(P):
                for c in range(V // 16):
                    accs[c] = accs[c] + rows_v[sl, k * P + r,
                                               pl.ds(c * 16, 16)]
            for c in range(8):
                acc_v[j * POS_CH + k, pl.ds(c * 16, 16)] = accs[c]
            return carry

        lax.fori_loop(0, POS_CH, pos_body, 0)
    pltpu.sync_copy(acc_v, out_hbm.at[pl.ds(wid * POS_W, POS_W)])


@functools.partial(
    pl.kernel,
    mesh=_sc_mesh(),
    out_type=jax.ShapeDtypeStruct((NW, ROW_W, 128), jnp.float32),
    compiler_params=pltpu.CompilerParams(use_tc_tiling_on_sc=False,
                                         needs_layout_passes=False),
    scratch_types=[
        pltpu.VMEM((ROW_W, 128), jnp.int32),
        pltpu.VMEM((NCK, POS_W, 128), jnp.float32),
        pltpu.VMEM((ROW_W, 128), jnp.float32),
    ],
)
def _sc_pick(midx_hbm, scores_hbm, out_hbm, midx_v, rows_v, out_v):
    wid = lax.axis_index("s") * NC + lax.axis_index("c")
    base = wid * POS_W
    pltpu.sync_copy(midx_hbm.at[wid], midx_v)
    for c in range(NCK):
        pltpu.sync_copy(scores_hbm.at[c, pl.ds(base, POS_W)], rows_v.at[c])

    def row_body(r, carry):
        for q in range(8):
            cvec = midx_v[r, pl.ds(q * 16, 16)]
            ckvec = lax.shift_right_logical(cvec, 7)
            colvec = lax.bitwise_and(cvec, 127)
            posvec = jnp.broadcast_to(r * 4 + q // 2, (16,)).astype(jnp.int32)
            out_v[r, pl.ds(q * 16, 16)] = plsc.load_gather(
                rows_v, [ckvec, posvec, colvec])
        return carry

    lax.fori_loop(0, ROW_W, row_body, 0)
    pltpu.sync_copy(out_v, out_hbm.at[wid])


def _tc_body(b_ref, w_ref, ow_ref, ob_ref, o_ref):
    bvec = b_ref[...]
    acc = jnp.zeros((BT, NMV), jnp.float32) + ob_ref[...]
    for h in range(V2):
        hid = jnp.dot(bvec, w_ref[h], preferred_element_type=jnp.float32)
        acc = acc + jnp.maximum(hid, 0.0) * ow_ref[h][None, :]
    for c in range(NCK):
        o_ref[c] = acc[:, c * 128:(c + 1) * 128]


def _tc_dense(b128, w, ow, ob):
    return pl.pallas_call(
        _tc_body,
        grid=(B // BT,),
        in_specs=[
            pl.BlockSpec((BT, 128), lambda i: (i, 0)),
            pl.BlockSpec((V2, 128, NMV), lambda i: (0, 0, 0)),
            pl.BlockSpec((V2, NMV), lambda i: (0, 0)),
            pl.BlockSpec((1, NMV), lambda i: (0, 0)),
        ],
        out_specs=pl.BlockSpec((NCK, BT, 128), lambda i: (0, i, 0)),
        out_shape=jax.ShapeDtypeStruct((NCK, B, 128), jnp.float32),
    )(b128, w, ow, ob)


@jax.jit
def kernel(piece_square_idx, move_idx, ps_vectors, move_vectors, ps_bias,
           bias2, output_layer, output_bias):
    psq = piece_square_idx.astype(jnp.int32).reshape(NW, NCH, ICH)
    midx = move_idx.astype(jnp.int32).reshape(NW, ROW_W, 128)
    psb128 = jnp.concatenate(
        [ps_bias, jnp.ones((1,), jnp.float32), jnp.zeros((63,), jnp.float32)])
    w = jnp.transpose(move_vectors, (2, 1, 0))
    b2 = jnp.transpose(bias2, (1, 0))
    w_aug = jnp.concatenate(
        [w, b2[:, None, :], jnp.zeros((V2, 63, NMV), jnp.float32)], axis=1)
    ow = jnp.transpose(output_layer, (1, 0))

    b128 = _sc_embed(psq, ps_vectors, psb128)
    scores = _tc_dense(b128, w_aug, ow, output_bias[None, :])
    return _sc_pick(midx, scores).reshape(B, M)
